# Initial kernel scaffold; baseline (speedup 1.0000x reference)
import functools

import jax
import jax.numpy as jnp
from jax import lax
from jax.experimental import pallas as pl
from jax.experimental.pallas import tpu as pltpu
from jax.experimental.pallas import tpu_sc as plsc

N = 10000
IN_C = 128
OUT_C = 128
K = 16
M = 2500
EPS = 1e-5

SUB = 8
LANES = 1280
NPAD = SUB * LANES

NW = 32
PW = 80
MPAD = NW * PW
CH = 8
NCHUNK = PW // CH
ROWS = CH * K

NEGINF = float("-inf")
POSINF = float("inf")
BIGI = jnp.int32(2**30)


def _fps_body(xr, yr, zr, sel_ref, sx_ref, sy_ref, sz_ref):
    X = xr[...]
    Y = yr[...]
    Z = zr[...]
    IDX = (lax.broadcasted_iota(jnp.int32, (SUB, LANES), 0) * LANES
           + lax.broadcasted_iota(jnp.int32, (SUB, LANES), 1))
    neg = jnp.float32(NEGINF)

    m0 = IDX == 0
    px = jnp.max(jnp.where(m0, X, neg))
    py = jnp.max(jnp.where(m0, Y, neg))
    pz = jnp.max(jnp.where(m0, Z, neg))
    sel_ref[0] = jnp.int32(0)
    sx_ref[0] = px
    sy_ref[0] = py
    sz_ref[0] = pz
    dists0 = jnp.where(IDX < N, jnp.float32(POSINF), neg)

    def body(i, carry):
        dists, px, py, pz = carry
        dx = X - px
        dy = Y - py
        dz = Z - pz
        d = dx * dx + dy * dy + dz * dz
        dists = jnp.minimum(dists, d)
        maxv = jnp.max(dists)
        nxt = jnp.min(jnp.where(dists == maxv, IDX, BIGI))
        sel_ref[i] = nxt
        m = IDX == nxt
        px = jnp.max(jnp.where(m, X, neg))
        py = jnp.max(jnp.where(m, Y, neg))
        pz = jnp.max(jnp.where(m, Z, neg))
        sx_ref[i] = px
        sy_ref[i] = py
        sz_ref[i] = pz
        return dists, px, py, pz

    lax.fori_loop(1, M, body, (dists0, px, py, pz))


def _fps(Xp, Yp, Zp):
    return pl.pallas_call(
        _fps_body,
        out_shape=[
            jax.ShapeDtypeStruct((M,), jnp.int32),
            jax.ShapeDtypeStruct((M,), jnp.float32),
            jax.ShapeDtypeStruct((M,), jnp.float32),
            jax.ShapeDtypeStruct((M,), jnp.float32),
        ],
        in_specs=[pl.BlockSpec(memory_space=pltpu.VMEM)] * 3,
        out_specs=[pl.BlockSpec(memory_space=pltpu.SMEM)] * 4,
    )(Xp, Yp, Zp)


RT = 400
NRT = N // RT


def _mlp_body(x_ref, wt_ref, b_ref, g_ref, be_ref, h_ref):
    wt = wt_ref[...]
    bb = b_ref[...]

    def p1(t, acc):
        s, ss = acc
        xt = x_ref[pl.ds(t * RT, RT), :]
        ht = jnp.dot(xt, wt, preferred_element_type=jnp.float32) + bb
        h_ref[pl.ds(t * RT, RT), :] = ht
        return (s + jnp.sum(ht, axis=0, keepdims=True),
                ss + jnp.sum(ht * ht, axis=0, keepdims=True))

    zero = jnp.zeros((1, OUT_C), jnp.float32)
    s, ss = lax.fori_loop(0, NRT, p1, (zero, zero))
    mean = s * (1.0 / N)
    var = ss * (1.0 / N) - mean * mean
    scale = g_ref[...] / jnp.sqrt(var + EPS)
    shift = be_ref[...] - mean * scale

    def p2(t, _):
        ht = h_ref[pl.ds(t * RT, RT), :]
        h_ref[pl.ds(t * RT, RT), :] = jnp.maximum(ht * scale + shift, 0.0)
        return 0

    lax.fori_loop(0, NRT, p2, 0)


def _mlp(x, Wt, b2, g2, be2):
    return pl.pallas_call(
        _mlp_body,
        out_shape=jax.ShapeDtypeStruct((N, OUT_C), jnp.float32),
        in_specs=[pl.BlockSpec(memory_space=pltpu.VMEM)] * 5,
        out_specs=pl.BlockSpec(memory_space=pltpu.VMEM),
    )(x, Wt, b2, g2, be2)


CB = 128
NCB = MPAD // CB
CT = 256
NCT = NPAD // CT


def _knn_body(px_ref, py_ref, pz_ref, yx_ref, yy_ref, yz_ref, nbr_ref, d2_ref):
    yx = yx_ref[...]
    yy = yy_ref[...]
    yz = yz_ref[...]
    sy = yx * yx + yy * yy + yz * yz
    inf = jnp.float32(POSINF)

    def dcompute(t, mn):
        Xc = px_ref[:, pl.ds(t * CT, CT)]
        Yc = py_ref[:, pl.ds(t * CT, CT)]
        Zc = pz_ref[:, pl.ds(t * CT, CT)]
        spc = Xc * Xc + Yc * Yc + Zc * Zc
        d2t = (sy + spc) - 2.0 * (yx * Xc + yy * Yc + yz * Zc)
        colid = t * CT + lax.broadcasted_iota(jnp.int32, (CB, CT), 1)
        d2t = jnp.where(colid < N, d2t, inf)
        d2_ref[:, pl.ds(t * CT, CT)] = d2t
        return jnp.minimum(mn, jnp.min(d2t, axis=1, keepdims=True))

    mn = lax.fori_loop(0, NCT, dcompute, jnp.full((CB, 1), inf, jnp.float32))

    for k in range(K):
        def targ(t, am, mn=mn):
            d2t = d2_ref[:, pl.ds(t * CT, CT)]
            colid = t * CT + lax.broadcasted_iota(jnp.int32, (CB, CT), 1)
            cand = jnp.where(d2t == mn, colid, BIGI)
            return jnp.minimum(am, jnp.min(cand, axis=1, keepdims=True))

        am = lax.fori_loop(0, NCT, targ, jnp.full((CB, 1), BIGI, jnp.int32))
        nbr_ref[:, k:k + 1] = am

        if k < K - 1:
            def tinv(t, mn2, am=am):
                d2t = d2_ref[:, pl.ds(t * CT, CT)]
                colid = t * CT + lax.broadcasted_iota(jnp.int32, (CB, CT), 1)
                d2t = jnp.where(colid == am, inf, d2t)
                d2_ref[:, pl.ds(t * CT, CT)] = d2t
                return jnp.minimum(mn2, jnp.min(d2t, axis=1, keepdims=True))

            mn = lax.fori_loop(0, NCT, tinv,
                               jnp.full((CB, 1), inf, jnp.float32))


def _knn(Xr, Yr, Zr, yxp, yyp, yzp):
    return pl.pallas_call(
        _knn_body,
        grid=(NCB,),
        out_shape=jax.ShapeDtypeStruct((MPAD, K), jnp.int32),
        in_specs=[
            pl.BlockSpec((1, NPAD), lambda t: (0, 0)),
            pl.BlockSpec((1, NPAD), lambda t: (0, 0)),
            pl.BlockSpec((1, NPAD), lambda t: (0, 0)),
            pl.BlockSpec((CB, 1), lambda t: (t, 0)),
            pl.BlockSpec((CB, 1), lambda t: (t, 0)),
            pl.BlockSpec((CB, 1), lambda t: (t, 0)),
        ],
        out_specs=pl.BlockSpec((CB, K), lambda t: (t, 0)),
        scratch_shapes=[pltpu.VMEM((CB, NPAD), jnp.float32)],
    )(Xr, Yr, Zr, yxp, yyp, yzp)


def _sc_body(col_hbm, h_hbm, sel_hbm, batch_hbm, out_hbm, sb_hbm,
             idx_v, rows_v, outc_v, sel_v, sb_v, batch_v, sem_g):
    cid = lax.axis_index("c")
    sid = lax.axis_index("s")
    wid = sid * 2 + cid
    base = wid * PW

    pltpu.sync_copy(col_hbm.at[wid], idx_v)
    pltpu.sync_copy(sel_hbm.at[pl.ds(base, PW)], sel_v)
    pltpu.sync_copy(batch_hbm, batch_v)

    def chunk(c, _):
        pltpu.async_copy(h_hbm.at[idx_v.at[c]], rows_v, sem_g).wait()
        for j in range(CH):
            for ch in range(OUT_C // 16):
                acc = rows_v[j * K, pl.ds(ch * 16, 16)]
                for t in range(1, K):
                    acc = jnp.maximum(acc, rows_v[j * K + t, pl.ds(ch * 16, 16)])
                outc_v[j, pl.ds(ch * 16, 16)] = acc
        pltpu.sync_copy(outc_v, out_hbm.at[pl.ds(base + c * CH, CH)])
        return 0

    lax.fori_loop(0, NCHUNK, chunk, 0)

    for i in range(PW // 16):
        sidx = sel_v[pl.ds(i * 16, 16)]
        sb_v[pl.ds(i * 16, 16)] = plsc.load_gather(batch_v, [sidx])
    pltpu.sync_copy(sb_v, sb_hbm.at[pl.ds(base, PW)])


def _sc_segmax(col3, h, selp, batch_i32):
    mesh = plsc.VectorSubcoreMesh(core_axis_name="c", subcore_axis_name="s")
    f = pl.kernel(
        _sc_body,
        out_type=[
            jax.ShapeDtypeStruct((MPAD, OUT_C), jnp.float32),
            jax.ShapeDtypeStruct((MPAD,), jnp.int32),
        ],
        mesh=mesh,
        scratch_types=[
            pltpu.VMEM((NCHUNK, ROWS), jnp.int32),
            pltpu.VMEM((ROWS, OUT_C), jnp.float32),
            pltpu.VMEM((CH, OUT_C), jnp.float32),
            pltpu.VMEM((PW,), jnp.int32),
            pltpu.VMEM((PW,), jnp.int32),
            pltpu.VMEM((N,), jnp.int32),
            pltpu.SemaphoreType.DMA,
        ],
    )
    return f(col3, h, selp, batch_i32)


def kernel(x, pos, batch, W, b, gamma, beta):
    posp = jnp.pad(pos.astype(jnp.float32), ((0, NPAD - N), (0, 0)))
    Xp = posp[:, 0].reshape(SUB, LANES)
    Yp = posp[:, 1].reshape(SUB, LANES)
    Zp = posp[:, 2].reshape(SUB, LANES)

    sel, sx, sy, sz = _fps(Xp, Yp, Zp)

    h = _mlp(x, W.T, b.reshape(1, OUT_C), gamma.reshape(1, OUT_C),
             beta.reshape(1, OUT_C))

    Xr = posp[:, 0].reshape(1, NPAD)
    Yr = posp[:, 1].reshape(1, NPAD)
    Zr = posp[:, 2].reshape(1, NPAD)
    yxp = jnp.pad(sx, (0, MPAD - M)).reshape(MPAD, 1)
    yyp = jnp.pad(sy, (0, MPAD - M)).reshape(MPAD, 1)
    yzp = jnp.pad(sz, (0, MPAD - M)).reshape(MPAD, 1)
    nbr = _knn(Xr, Yr, Zr, yxp, yyp, yzp)

    col3 = nbr.reshape(NW, NCHUNK, ROWS)
    selp = jnp.pad(sel, (0, MPAD - M))
    batch_i32 = batch.astype(jnp.int32)
    out_full, sb_full = _sc_segmax(col3, h, selp, batch_i32)

    out = out_full[:M]
    sub_pos = jnp.stack([sx, sy, sz], axis=1)
    sub_batch = sb_full[:M].astype(batch.dtype)
    return (out, sub_pos, sub_batch)


# speedup vs baseline: 5.3736x; 5.3736x over previous
import functools

import jax
import jax.numpy as jnp
from jax import lax
from jax.experimental import pallas as pl
from jax.experimental.pallas import tpu as pltpu
from jax.experimental.pallas import tpu_sc as plsc

N = 10000
IN_C = 128
OUT_C = 128
K = 16
M = 2500
EPS = 1e-5

SUB = 8
LANES = 1280
NPAD = SUB * LANES

NW = 32
PW = 80
MPAD = NW * PW
CH = 8
NCHUNK = PW // CH
ROWS = CH * K

NEGINF = float("-inf")
POSINF = float("inf")
BIGI = 2**30


INT32_MIN = -(2**31)


def _fps_body(xr, yr, zr, br, sx_ref, sy_ref, sz_ref, sb_ref):
    X = xr[...]
    Y = yr[...]
    Z = zr[...]
    B = br[...]
    IDX = (lax.broadcasted_iota(jnp.int32, (SUB, LANES), 0) * LANES
           + lax.broadcasted_iota(jnp.int32, (SUB, LANES), 1))
    neg = jnp.float32(NEGINF)

    m0 = IDX == 0
    px = jnp.max(jnp.where(m0, X, neg))
    py = jnp.max(jnp.where(m0, Y, neg))
    pz = jnp.max(jnp.where(m0, Z, neg))
    sx_ref[0] = px
    sy_ref[0] = py
    sz_ref[0] = pz
    sb_ref[0] = jnp.max(jnp.where(m0, B, INT32_MIN))
    dists0 = jnp.where(IDX < N, jnp.float32(POSINF), neg)

    def body(i, carry):
        dists, px, py, pz = carry
        dx = X - px
        dy = Y - py
        dz = Z - pz
        d = dx * dx + dy * dy + dz * dz
        dists = jnp.minimum(dists, d)
        maxv = jnp.max(dists)
        nxt = jnp.min(jnp.where(dists == maxv, IDX, BIGI))
        m = IDX == nxt
        px = jnp.max(jnp.where(m, X, neg))
        py = jnp.max(jnp.where(m, Y, neg))
        pz = jnp.max(jnp.where(m, Z, neg))
        sx_ref[i] = px
        sy_ref[i] = py
        sz_ref[i] = pz
        sb_ref[i] = jnp.max(jnp.where(m, B, INT32_MIN))
        return dists, px, py, pz

    lax.fori_loop(1, M, body, (dists0, px, py, pz))


def _fps(Xp, Yp, Zp, Bp):
    return pl.pallas_call(
        _fps_body,
        out_shape=[
            jax.ShapeDtypeStruct((M,), jnp.float32),
            jax.ShapeDtypeStruct((M,), jnp.float32),
            jax.ShapeDtypeStruct((M,), jnp.float32),
            jax.ShapeDtypeStruct((M,), jnp.int32),
        ],
        in_specs=[pl.BlockSpec(memory_space=pltpu.VMEM)] * 4,
        out_specs=[pl.BlockSpec(memory_space=pltpu.SMEM)] * 4,
    )(Xp, Yp, Zp, Bp)


RT = 400
NRT = N // RT


def _mlp_body(x_ref, wt_ref, b_ref, g_ref, be_ref, h_ref):
    wt = wt_ref[...]
    bb = b_ref[...]

    def p1(t, acc):
        s, ss = acc
        xt = x_ref[pl.ds(t * RT, RT), :]
        ht = jnp.dot(xt, wt, preferred_element_type=jnp.float32) + bb
        h_ref[pl.ds(t * RT, RT), :] = ht
        return (s + jnp.sum(ht, axis=0, keepdims=True),
                ss + jnp.sum(ht * ht, axis=0, keepdims=True))

    zero = jnp.zeros((1, OUT_C), jnp.float32)
    s, ss = lax.fori_loop(0, NRT, p1, (zero, zero))
    mean = s * (1.0 / N)
    var = ss * (1.0 / N) - mean * mean
    scale = g_ref[...] / jnp.sqrt(var + EPS)
    shift = be_ref[...] - mean * scale

    def p2(t, _):
        ht = h_ref[pl.ds(t * RT, RT), :]
        h_ref[pl.ds(t * RT, RT), :] = jnp.maximum(ht * scale + shift, 0.0)
        return 0

    lax.fori_loop(0, NRT, p2, 0)


def _mlp(x, Wt, b2, g2, be2):
    return pl.pallas_call(
        _mlp_body,
        out_shape=jax.ShapeDtypeStruct((N, OUT_C), jnp.float32),
        in_specs=[pl.BlockSpec(memory_space=pltpu.VMEM)] * 5,
        out_specs=pl.BlockSpec(memory_space=pltpu.VMEM),
    )(x, Wt, b2, g2, be2)


CB = 128
NCB = MPAD // CB
CT = 256
NCT = NPAD // CT


def _knn_body(px_ref, py_ref, pz_ref, yx_ref, yy_ref, yz_ref, nbr_ref, d2_ref):
    yx = yx_ref[...]
    yy = yy_ref[...]
    yz = yz_ref[...]
    sy = yx * yx + yy * yy + yz * yz
    def _r(v):
        return v.astype(jnp.bfloat16).astype(jnp.float32)

    yxb = _r(yx)
    yyb = _r(yy)
    yzb = _r(yz)
    inf = jnp.float32(POSINF)

    def dcompute(t, mn):
        Xc = px_ref[:, pl.ds(t * CT, CT)]
        Yc = py_ref[:, pl.ds(t * CT, CT)]
        Zc = pz_ref[:, pl.ds(t * CT, CT)]
        spc = Xc * Xc + Yc * Yc + Zc * Zc
        d2t = (sy + spc) - 2.0 * (yxb * _r(Xc) + yyb * _r(Yc) + yzb * _r(Zc))
        colid = t * CT + lax.broadcasted_iota(jnp.int32, (CB, CT), 1)
        d2t = jnp.where(colid < N, d2t, inf)
        d2_ref[:, pl.ds(t * CT, CT)] = d2t
        return jnp.minimum(mn, jnp.min(d2t, axis=1, keepdims=True))

    mn = lax.fori_loop(0, NCT, dcompute, jnp.full((CB, 1), inf, jnp.float32))

    for k in range(K):
        def targ(t, am, mn=mn):
            d2t = d2_ref[:, pl.ds(t * CT, CT)]
            colid = t * CT + lax.broadcasted_iota(jnp.int32, (CB, CT), 1)
            cand = jnp.where(d2t == mn, colid, BIGI)
            return jnp.minimum(am, jnp.min(cand, axis=1, keepdims=True))

        am = lax.fori_loop(0, NCT, targ, jnp.full((CB, 1), BIGI, jnp.int32))
        nbr_ref[:, k:k + 1] = am

        if k < K - 1:
            def tinv(t, mn2, am=am):
                d2t = d2_ref[:, pl.ds(t * CT, CT)]
                colid = t * CT + lax.broadcasted_iota(jnp.int32, (CB, CT), 1)
                d2t = jnp.where(colid == am, inf, d2t)
                d2_ref[:, pl.ds(t * CT, CT)] = d2t
                return jnp.minimum(mn2, jnp.min(d2t, axis=1, keepdims=True))

            mn = lax.fori_loop(0, NCT, tinv,
                               jnp.full((CB, 1), inf, jnp.float32))


def _knn(Xr, Yr, Zr, yxp, yyp, yzp):
    return pl.pallas_call(
        _knn_body,
        grid=(NCB,),
        out_shape=jax.ShapeDtypeStruct((MPAD, K), jnp.int32),
        in_specs=[
            pl.BlockSpec((1, NPAD), lambda t: (0, 0)),
            pl.BlockSpec((1, NPAD), lambda t: (0, 0)),
            pl.BlockSpec((1, NPAD), lambda t: (0, 0)),
            pl.BlockSpec((CB, 1), lambda t: (t, 0)),
            pl.BlockSpec((CB, 1), lambda t: (t, 0)),
            pl.BlockSpec((CB, 1), lambda t: (t, 0)),
        ],
        out_specs=pl.BlockSpec((CB, K), lambda t: (t, 0)),
        scratch_shapes=[pltpu.VMEM((CB, NPAD), jnp.float32)],
    )(Xr, Yr, Zr, yxp, yyp, yzp)


def _sc_body(col_hbm, h_hbm, out_hbm, idx_v, rows_v, outc_v, sem_g):
    cid = lax.axis_index("c")
    sid = lax.axis_index("s")
    wid = sid * 2 + cid
    base = wid * PW

    pltpu.sync_copy(col_hbm.at[wid], idx_v)

    def chunk(c, _):
        pltpu.async_copy(h_hbm.at[idx_v.at[c]], rows_v, sem_g).wait()
        for j in range(CH):
            for ch in range(OUT_C // 16):
                acc = rows_v[j * K, pl.ds(ch * 16, 16)]
                for t in range(1, K):
                    acc = jnp.maximum(acc, rows_v[j * K + t, pl.ds(ch * 16, 16)])
                outc_v[j, pl.ds(ch * 16, 16)] = acc
        pltpu.sync_copy(outc_v, out_hbm.at[pl.ds(base + c * CH, CH)])
        return 0

    lax.fori_loop(0, NCHUNK, chunk, 0)


def _sc_segmax(col3, h):
    mesh = plsc.VectorSubcoreMesh(core_axis_name="c", subcore_axis_name="s")
    f = pl.kernel(
        _sc_body,
        out_type=jax.ShapeDtypeStruct((MPAD, OUT_C), jnp.float32),
        mesh=mesh,
        scratch_types=[
            pltpu.VMEM((NCHUNK, ROWS), jnp.int32),
            pltpu.VMEM((ROWS, OUT_C), jnp.float32),
            pltpu.VMEM((CH, OUT_C), jnp.float32),
            pltpu.SemaphoreType.DMA,
        ],
    )
    return f(col3, h)


def kernel(x, pos, batch, W, b, gamma, beta):
    posp = jnp.pad(pos.astype(jnp.float32), ((0, NPAD - N), (0, 0)))
    Xp = posp[:, 0].reshape(SUB, LANES)
    Yp = posp[:, 1].reshape(SUB, LANES)
    Zp = posp[:, 2].reshape(SUB, LANES)
    Bp = jnp.pad(batch.astype(jnp.int32), (0, NPAD - N)).reshape(SUB, LANES)

    sx, sy, sz, sb = _fps(Xp, Yp, Zp, Bp)

    h = _mlp(x, W.T, b.reshape(1, OUT_C), gamma.reshape(1, OUT_C),
             beta.reshape(1, OUT_C))

    Xr = posp[:, 0].reshape(1, NPAD)
    Yr = posp[:, 1].reshape(1, NPAD)
    Zr = posp[:, 2].reshape(1, NPAD)
    yxp = jnp.pad(sx, (0, MPAD - M)).reshape(MPAD, 1)
    yyp = jnp.pad(sy, (0, MPAD - M)).reshape(MPAD, 1)
    yzp = jnp.pad(sz, (0, MPAD - M)).reshape(MPAD, 1)
    nbr = _knn(Xr, Yr, Zr, yxp, yyp, yzp)

    col3 = nbr.reshape(NW, NCHUNK, ROWS)
    out_full = _sc_segmax(col3, h)

    out = out_full[:M]
    sub_pos = jnp.stack([sx, sy, sz], axis=1)
    sub_batch = sb.astype(batch.dtype)
    return (out, sub_pos, sub_batch)


# speedup vs baseline: 12.0680x; 2.2458x over previous
import functools

import jax
import jax.numpy as jnp
from jax import lax
from jax.experimental import pallas as pl
from jax.experimental.pallas import tpu as pltpu
from jax.experimental.pallas import tpu_sc as plsc

N = 10000
IN_C = 128
OUT_C = 128
K = 16
M = 2500
EPS = 1e-5

SUB = 8
LANES = 1280
NPAD = SUB * LANES

NW = 32
PW = 80
MPAD = NW * PW
CH = 8
NCHUNK = PW // CH
ROWS = CH * K

NEGINF = float("-inf")
POSINF = float("inf")
BIGI = 2**30


INT32_MIN = -(2**31)


def _fps_body(xr, yr, zr, br, sx_ref, sy_ref, sz_ref, sb_ref):
    X = xr[...]
    Y = yr[...]
    Z = zr[...]
    B = br[...]
    IDX = (lax.broadcasted_iota(jnp.int32, (SUB, LANES), 0) * LANES
           + lax.broadcasted_iota(jnp.int32, (SUB, LANES), 1))
    neg = jnp.float32(NEGINF)

    def extract(m):
        px = jnp.max(jnp.where(m, X, neg), keepdims=True)
        py = jnp.max(jnp.where(m, Y, neg), keepdims=True)
        pz = jnp.max(jnp.where(m, Z, neg), keepdims=True)
        pb = jnp.max(jnp.where(m, B, INT32_MIN), keepdims=True)
        return px, py, pz, pb

    px, py, pz, pb = extract(IDX == 0)
    sx_ref[0] = px[0, 0]
    sy_ref[0] = py[0, 0]
    sz_ref[0] = pz[0, 0]
    sb_ref[0] = pb[0, 0]
    dists0 = jnp.where(IDX < N, jnp.float32(POSINF), 0.0)

    def body(i, carry):
        dists, px, py, pz = carry
        dx = X - px
        dy = Y - py
        dz = Z - pz
        d = dx * dx + dy * dy + dz * dz
        dists = jnp.minimum(dists, d)
        maxv = jnp.max(dists, keepdims=True)
        m1 = dists == maxv
        cx, cy, cz, cb = extract(m1)
        nxtv = jnp.min(jnp.where(m1, IDX, BIGI), keepdims=True)
        mxiv = jnp.max(jnp.where(m1, IDX, -1), keepdims=True)

        def tie_fix(_):
            return extract(IDX == nxtv)

        px, py, pz, pb = lax.cond(nxtv[0, 0] == mxiv[0, 0],
                                  lambda _: (cx, cy, cz, cb), tie_fix, 0)
        sx_ref[i] = px[0, 0]
        sy_ref[i] = py[0, 0]
        sz_ref[i] = pz[0, 0]
        sb_ref[i] = pb[0, 0]
        return dists, px, py, pz

    lax.fori_loop(1, M, body, (dists0, px, py, pz))


def _fps(Xp, Yp, Zp, Bp):
    return pl.pallas_call(
        _fps_body,
        out_shape=[
            jax.ShapeDtypeStruct((M,), jnp.float32),
            jax.ShapeDtypeStruct((M,), jnp.float32),
            jax.ShapeDtypeStruct((M,), jnp.float32),
            jax.ShapeDtypeStruct((M,), jnp.int32),
        ],
        in_specs=[pl.BlockSpec(memory_space=pltpu.VMEM)] * 4,
        out_specs=[pl.BlockSpec(memory_space=pltpu.SMEM)] * 4,
    )(Xp, Yp, Zp, Bp)


RT = 400
NRT = N // RT


def _mlp_body(x_ref, wt_ref, b_ref, g_ref, be_ref, h_ref):
    wt = wt_ref[...]
    bb = b_ref[...]

    def p1(t, acc):
        s, ss = acc
        xt = x_ref[pl.ds(t * RT, RT), :]
        ht = jnp.dot(xt, wt, preferred_element_type=jnp.float32) + bb
        h_ref[pl.ds(t * RT, RT), :] = ht
        return (s + jnp.sum(ht, axis=0, keepdims=True),
                ss + jnp.sum(ht * ht, axis=0, keepdims=True))

    zero = jnp.zeros((1, OUT_C), jnp.float32)
    s, ss = lax.fori_loop(0, NRT, p1, (zero, zero))
    mean = s * (1.0 / N)
    var = ss * (1.0 / N) - mean * mean
    scale = g_ref[...] / jnp.sqrt(var + EPS)
    shift = be_ref[...] - mean * scale

    def p2(t, _):
        ht = h_ref[pl.ds(t * RT, RT), :]
        h_ref[pl.ds(t * RT, RT), :] = jnp.maximum(ht * scale + shift, 0.0)
        return 0

    lax.fori_loop(0, NRT, p2, 0)


def _mlp(x, Wt, b2, g2, be2):
    return pl.pallas_call(
        _mlp_body,
        out_shape=jax.ShapeDtypeStruct((N, OUT_C), jnp.float32),
        in_specs=[pl.BlockSpec(memory_space=pltpu.VMEM)] * 5,
        out_specs=pl.BlockSpec(memory_space=pltpu.VMEM),
    )(x, Wt, b2, g2, be2)


CB = 128
NCB = MPAD // CB
CT = 1024
NCT = NPAD // CT


def _knn_body(px_ref, py_ref, pz_ref, yx_ref, yy_ref, yz_ref, nbr_ref, d2_ref):
    yx = yx_ref[...]
    yy = yy_ref[...]
    yz = yz_ref[...]
    sy = yx * yx + yy * yy + yz * yz
    def _r(v):
        return v.astype(jnp.bfloat16).astype(jnp.float32)

    yxb = _r(yx)
    yyb = _r(yy)
    yzb = _r(yz)
    inf = jnp.float32(POSINF)

    def dcompute(t, mn):
        Xc = px_ref[:, pl.ds(t * CT, CT)]
        Yc = py_ref[:, pl.ds(t * CT, CT)]
        Zc = pz_ref[:, pl.ds(t * CT, CT)]
        spc = Xc * Xc + Yc * Yc + Zc * Zc
        d2t = (sy + spc) - 2.0 * (yxb * _r(Xc) + yyb * _r(Yc) + yzb * _r(Zc))
        colid = t * CT + lax.broadcasted_iota(jnp.int32, (CB, CT), 1)
        d2t = jnp.where(colid < N, d2t, inf)
        d2_ref[:, pl.ds(t * CT, CT)] = d2t
        return jnp.minimum(mn, jnp.min(d2t, axis=1, keepdims=True))

    mn = lax.fori_loop(0, NCT, dcompute, jnp.full((CB, 1), inf, jnp.float32))
    am = jnp.full((CB, 1), -1, jnp.int32)

    for k in range(K):
        def fused(t, carry, mn=mn, am=am):
            aidx, gmx, smin = carry
            d2t = d2_ref[:, pl.ds(t * CT, CT)]
            colid = t * CT + lax.broadcasted_iota(jnp.int32, (CB, CT), 1)
            e1 = d2t == mn
            aidx = jnp.minimum(aidx, jnp.min(
                jnp.where(e1 & (colid > am), colid, BIGI),
                axis=1, keepdims=True))
            gmx = jnp.maximum(gmx, jnp.max(
                jnp.where(e1, colid, -1), axis=1, keepdims=True))
            smin = jnp.minimum(smin, jnp.min(
                jnp.where(d2t > mn, d2t, inf), axis=1, keepdims=True))
            return aidx, gmx, smin

        aidx, gmx, smin = lax.fori_loop(
            0, NCT, fused,
            (jnp.full((CB, 1), BIGI, jnp.int32),
             jnp.full((CB, 1), -1, jnp.int32),
             jnp.full((CB, 1), inf, jnp.float32)))
        nbr_ref[:, k:k + 1] = aidx

        if k < K - 1:
            more = aidx < gmx
            mn = jnp.where(more, mn, smin)
            am = jnp.where(more, aidx, -1)


def _knn(Xr, Yr, Zr, yxp, yyp, yzp):
    return pl.pallas_call(
        _knn_body,
        grid=(NCB,),
        out_shape=jax.ShapeDtypeStruct((MPAD, K), jnp.int32),
        in_specs=[
            pl.BlockSpec((1, NPAD), lambda t: (0, 0)),
            pl.BlockSpec((1, NPAD), lambda t: (0, 0)),
            pl.BlockSpec((1, NPAD), lambda t: (0, 0)),
            pl.BlockSpec((CB, 1), lambda t: (t, 0)),
            pl.BlockSpec((CB, 1), lambda t: (t, 0)),
            pl.BlockSpec((CB, 1), lambda t: (t, 0)),
        ],
        out_specs=pl.BlockSpec((CB, K), lambda t: (t, 0)),
        scratch_shapes=[pltpu.VMEM((CB, NPAD), jnp.float32)],
    )(Xr, Yr, Zr, yxp, yyp, yzp)


def _sc_body(col_hbm, h_hbm, out_hbm, idx_v, rows_v, outc_v, sem_g):
    cid = lax.axis_index("c")
    sid = lax.axis_index("s")
    wid = sid * 2 + cid
    base = wid * PW

    pltpu.sync_copy(col_hbm.at[wid], idx_v)

    def chunk(c, _):
        pltpu.async_copy(h_hbm.at[idx_v.at[c]], rows_v, sem_g).wait()
        for j in range(CH):
            for ch in range(OUT_C // 16):
                acc = rows_v[j * K, pl.ds(ch * 16, 16)]
                for t in range(1, K):
                    acc = jnp.maximum(acc, rows_v[j * K + t, pl.ds(ch * 16, 16)])
                outc_v[j, pl.ds(ch * 16, 16)] = acc
        pltpu.sync_copy(outc_v, out_hbm.at[pl.ds(base + c * CH, CH)])
        return 0

    lax.fori_loop(0, NCHUNK, chunk, 0)


def _sc_segmax(col3, h):
    mesh = plsc.VectorSubcoreMesh(core_axis_name="c", subcore_axis_name="s")
    f = pl.kernel(
        _sc_body,
        out_type=jax.ShapeDtypeStruct((MPAD, OUT_C), jnp.float32),
        mesh=mesh,
        scratch_types=[
            pltpu.VMEM((NCHUNK, ROWS), jnp.int32),
            pltpu.VMEM((ROWS, OUT_C), jnp.float32),
            pltpu.VMEM((CH, OUT_C), jnp.float32),
            pltpu.SemaphoreType.DMA,
        ],
    )
    return f(col3, h)


def kernel(x, pos, batch, W, b, gamma, beta):
    posp = jnp.pad(pos.astype(jnp.float32), ((0, NPAD - N), (0, 0)))
    Xp = posp[:, 0].reshape(SUB, LANES)
    Yp = posp[:, 1].reshape(SUB, LANES)
    Zp = posp[:, 2].reshape(SUB, LANES)
    Bp = jnp.pad(batch.astype(jnp.int32), (0, NPAD - N)).reshape(SUB, LANES)

    sx, sy, sz, sb = _fps(Xp, Yp, Zp, Bp)

    h = _mlp(x, W.T, b.reshape(1, OUT_C), gamma.reshape(1, OUT_C),
             beta.reshape(1, OUT_C))

    Xr = posp[:, 0].reshape(1, NPAD)
    Yr = posp[:, 1].reshape(1, NPAD)
    Zr = posp[:, 2].reshape(1, NPAD)
    yxp = jnp.pad(sx, (0, MPAD - M)).reshape(MPAD, 1)
    yyp = jnp.pad(sy, (0, MPAD - M)).reshape(MPAD, 1)
    yzp = jnp.pad(sz, (0, MPAD - M)).reshape(MPAD, 1)
    nbr = _knn(Xr, Yr, Zr, yxp, yyp, yzp)

    col3 = nbr.reshape(NW, NCHUNK, ROWS)
    out_full = _sc_segmax(col3, h)

    out = out_full[:M]
    sub_pos = jnp.stack([sx, sy, sz], axis=1)
    sub_batch = sb.astype(batch.dtype)
    return (out, sub_pos, sub_batch)


# speedup vs baseline: 12.1494x; 1.0068x over previous
import functools

import jax
import jax.numpy as jnp
from jax import lax
from jax.experimental import pallas as pl
from jax.experimental.pallas import tpu as pltpu
from jax.experimental.pallas import tpu_sc as plsc

N = 10000
IN_C = 128
OUT_C = 128
K = 16
M = 2500
EPS = 1e-5

SUB = 8
LANES = 1280
NPAD = SUB * LANES

NW = 32
PW = 80
MPAD = NW * PW
CH = 8
NCHUNK = PW // CH
ROWS = CH * K

NEGINF = float("-inf")
POSINF = float("inf")
BIGI = 2**30


INT32_MIN = -(2**31)


def _fps_body(xr, yr, zr, br, sx_ref, sy_ref, sz_ref, sb_ref):
    X = xr[...]
    Y = yr[...]
    Z = zr[...]
    B = br[...]
    IDX = (lax.broadcasted_iota(jnp.int32, (SUB, LANES), 0) * LANES
           + lax.broadcasted_iota(jnp.int32, (SUB, LANES), 1))
    neg = jnp.float32(NEGINF)

    def extract(m):
        px = jnp.max(jnp.where(m, X, neg), keepdims=True)
        py = jnp.max(jnp.where(m, Y, neg), keepdims=True)
        pz = jnp.max(jnp.where(m, Z, neg), keepdims=True)
        pb = jnp.max(jnp.where(m, B, INT32_MIN), keepdims=True)
        return px, py, pz, pb

    px, py, pz, pb = extract(IDX == 0)
    sx_ref[0] = px[0, 0]
    sy_ref[0] = py[0, 0]
    sz_ref[0] = pz[0, 0]
    sb_ref[0] = pb[0, 0]
    dists0 = jnp.where(IDX < N, jnp.float32(POSINF), 0.0)

    def body(i, carry):
        dists, px, py, pz = carry
        dx = X - px
        dy = Y - py
        dz = Z - pz
        d = (dx * dx + dz * dz) + dy * dy
        dists = jnp.minimum(dists, d)
        maxv = jnp.max(dists, keepdims=True)
        m1 = dists == maxv
        cx, cy, cz, cb = extract(m1)
        nxtv = jnp.min(jnp.where(m1, IDX, BIGI), keepdims=True)
        mxiv = jnp.max(jnp.where(m1, IDX, -1), keepdims=True)

        def tie_fix(_):
            return extract(IDX == nxtv)

        px, py, pz, pb = lax.cond(nxtv[0, 0] == mxiv[0, 0],
                                  lambda _: (cx, cy, cz, cb), tie_fix, 0)
        sx_ref[i] = px[0, 0]
        sy_ref[i] = py[0, 0]
        sz_ref[i] = pz[0, 0]
        sb_ref[i] = pb[0, 0]
        return dists, px, py, pz

    lax.fori_loop(1, M, body, (dists0, px, py, pz))


def _fps(Xp, Yp, Zp, Bp):
    return pl.pallas_call(
        _fps_body,
        out_shape=[
            jax.ShapeDtypeStruct((M,), jnp.float32),
            jax.ShapeDtypeStruct((M,), jnp.float32),
            jax.ShapeDtypeStruct((M,), jnp.float32),
            jax.ShapeDtypeStruct((M,), jnp.int32),
        ],
        in_specs=[pl.BlockSpec(memory_space=pltpu.VMEM)] * 4,
        out_specs=[pl.BlockSpec(memory_space=pltpu.SMEM)] * 4,
    )(Xp, Yp, Zp, Bp)


RT = 400
NRT = N // RT


def _mlp_body(x_ref, wt_ref, b_ref, g_ref, be_ref, h_ref):
    wt = wt_ref[...]
    bb = b_ref[...]

    def p1(t, acc):
        s, ss = acc
        xt = x_ref[pl.ds(t * RT, RT), :]
        ht = jnp.dot(xt, wt, preferred_element_type=jnp.float32) + bb
        h_ref[pl.ds(t * RT, RT), :] = ht
        return (s + jnp.sum(ht, axis=0, keepdims=True),
                ss + jnp.sum(ht * ht, axis=0, keepdims=True))

    zero = jnp.zeros((1, OUT_C), jnp.float32)
    s, ss = lax.fori_loop(0, NRT, p1, (zero, zero))
    mean = s * (1.0 / N)
    var = ss * (1.0 / N) - mean * mean
    scale = g_ref[...] / jnp.sqrt(var + EPS)
    shift = be_ref[...] - mean * scale

    def p2(t, _):
        ht = h_ref[pl.ds(t * RT, RT), :]
        h_ref[pl.ds(t * RT, RT), :] = jnp.maximum(ht * scale + shift, 0.0)
        return 0

    lax.fori_loop(0, NRT, p2, 0)


def _mlp(x, Wt, b2, g2, be2):
    return pl.pallas_call(
        _mlp_body,
        out_shape=jax.ShapeDtypeStruct((N, OUT_C), jnp.float32),
        in_specs=[pl.BlockSpec(memory_space=pltpu.VMEM)] * 5,
        out_specs=pl.BlockSpec(memory_space=pltpu.VMEM),
    )(x, Wt, b2, g2, be2)


CB = 128
NCB = MPAD // CB
CT = 1024
NCT = NPAD // CT


def _knn_body(px_ref, py_ref, pz_ref, yx_ref, yy_ref, yz_ref, nbr_ref, d2_ref):
    yx = yx_ref[...]
    yy = yy_ref[...]
    yz = yz_ref[...]
    sy = (yx * yx + yz * yz) + yy * yy
    def _r(v):
        return v.astype(jnp.bfloat16).astype(jnp.float32)

    yxb = _r(yx)
    yyb = _r(yy)
    yzb = _r(yz)
    inf = jnp.float32(POSINF)

    def dcompute(t, mn):
        Xc = px_ref[:, pl.ds(t * CT, CT)]
        Yc = py_ref[:, pl.ds(t * CT, CT)]
        Zc = pz_ref[:, pl.ds(t * CT, CT)]
        spc = (Xc * Xc + Zc * Zc) + Yc * Yc
        d2t = (sy + spc) - 2.0 * (yxb * _r(Xc) + yyb * _r(Yc) + yzb * _r(Zc))
        colid = t * CT + lax.broadcasted_iota(jnp.int32, (CB, CT), 1)
        d2t = jnp.where(colid < N, d2t, inf)
        d2_ref[:, pl.ds(t * CT, CT)] = d2t
        return jnp.minimum(mn, jnp.min(d2t, axis=1, keepdims=True))

    mn = lax.fori_loop(0, NCT, dcompute, jnp.full((CB, 1), inf, jnp.float32))
    am = jnp.full((CB, 1), -1, jnp.int32)

    for k in range(K):
        def fused(t, carry, mn=mn, am=am):
            aidx, gmx, smin = carry
            d2t = d2_ref[:, pl.ds(t * CT, CT)]
            colid = t * CT + lax.broadcasted_iota(jnp.int32, (CB, CT), 1)
            e1 = d2t == mn
            aidx = jnp.minimum(aidx, jnp.min(
                jnp.where(e1 & (colid > am), colid, BIGI),
                axis=1, keepdims=True))
            gmx = jnp.maximum(gmx, jnp.max(
                jnp.where(e1, colid, -1), axis=1, keepdims=True))
            smin = jnp.minimum(smin, jnp.min(
                jnp.where(d2t > mn, d2t, inf), axis=1, keepdims=True))
            return aidx, gmx, smin

        aidx, gmx, smin = lax.fori_loop(
            0, NCT, fused,
            (jnp.full((CB, 1), BIGI, jnp.int32),
             jnp.full((CB, 1), -1, jnp.int32),
             jnp.full((CB, 1), inf, jnp.float32)))
        nbr_ref[:, k:k + 1] = aidx

        if k < K - 1:
            more = aidx < gmx
            mn = jnp.where(more, mn, smin)
            am = jnp.where(more, aidx, -1)


def _knn(Xr, Yr, Zr, yxp, yyp, yzp):
    return pl.pallas_call(
        _knn_body,
        grid=(NCB,),
        out_shape=jax.ShapeDtypeStruct((MPAD, K), jnp.int32),
        in_specs=[
            pl.BlockSpec((1, NPAD), lambda t: (0, 0)),
            pl.BlockSpec((1, NPAD), lambda t: (0, 0)),
            pl.BlockSpec((1, NPAD), lambda t: (0, 0)),
            pl.BlockSpec((CB, 1), lambda t: (t, 0)),
            pl.BlockSpec((CB, 1), lambda t: (t, 0)),
            pl.BlockSpec((CB, 1), lambda t: (t, 0)),
        ],
        out_specs=pl.BlockSpec((CB, K), lambda t: (t, 0)),
        scratch_shapes=[pltpu.VMEM((CB, NPAD), jnp.float32)],
    )(Xr, Yr, Zr, yxp, yyp, yzp)


def _sc_body(col_hbm, h_hbm, out_hbm, idx_v, rows_v, outc_v, sem_g):
    cid = lax.axis_index("c")
    sid = lax.axis_index("s")
    wid = sid * 2 + cid
    base = wid * PW

    pltpu.sync_copy(col_hbm.at[wid], idx_v)

    def chunk(c, _):
        pltpu.async_copy(h_hbm.at[idx_v.at[c]], rows_v, sem_g).wait()
        for j in range(CH):
            for ch in range(OUT_C // 16):
                acc = rows_v[j * K, pl.ds(ch * 16, 16)]
                for t in range(1, K):
                    acc = jnp.maximum(acc, rows_v[j * K + t, pl.ds(ch * 16, 16)])
                outc_v[j, pl.ds(ch * 16, 16)] = acc
        pltpu.sync_copy(outc_v, out_hbm.at[pl.ds(base + c * CH, CH)])
        return 0

    lax.fori_loop(0, NCHUNK, chunk, 0)


def _sc_segmax(col3, h):
    mesh = plsc.VectorSubcoreMesh(core_axis_name="c", subcore_axis_name="s")
    f = pl.kernel(
        _sc_body,
        out_type=jax.ShapeDtypeStruct((MPAD, OUT_C), jnp.float32),
        mesh=mesh,
        scratch_types=[
            pltpu.VMEM((NCHUNK, ROWS), jnp.int32),
            pltpu.VMEM((ROWS, OUT_C), jnp.float32),
            pltpu.VMEM((CH, OUT_C), jnp.float32),
            pltpu.SemaphoreType.DMA,
        ],
    )
    return f(col3, h)


def kernel(x, pos, batch, W, b, gamma, beta):
    posp = jnp.pad(pos.astype(jnp.float32), ((0, NPAD - N), (0, 0)))
    Xp = posp[:, 0].reshape(SUB, LANES)
    Yp = posp[:, 1].reshape(SUB, LANES)
    Zp = posp[:, 2].reshape(SUB, LANES)
    Bp = jnp.pad(batch.astype(jnp.int32), (0, NPAD - N)).reshape(SUB, LANES)

    sx, sy, sz, sb = _fps(Xp, Yp, Zp, Bp)

    h = _mlp(x, W.T, b.reshape(1, OUT_C), gamma.reshape(1, OUT_C),
             beta.reshape(1, OUT_C))

    Xr = posp[:, 0].reshape(1, NPAD)
    Yr = posp[:, 1].reshape(1, NPAD)
    Zr = posp[:, 2].reshape(1, NPAD)
    yxp = jnp.pad(sx, (0, MPAD - M)).reshape(MPAD, 1)
    yyp = jnp.pad(sy, (0, MPAD - M)).reshape(MPAD, 1)
    yzp = jnp.pad(sz, (0, MPAD - M)).reshape(MPAD, 1)
    nbr = _knn(Xr, Yr, Zr, yxp, yyp, yzp)

    col3 = nbr.reshape(NW, NCHUNK, ROWS)
    out_full = _sc_segmax(col3, h)

    out = out_full[:M]
    sub_pos = jnp.stack([sx, sy, sz], axis=1)
    sub_batch = sb.astype(batch.dtype)
    return (out, sub_pos, sub_batch)
